# Initial kernel scaffold; baseline (speedup 1.0000x reference)
import jax
import jax.numpy as jnp
from jax.experimental import pallas as pl


def kernel(x, edge_index, edge_type, W, root, bias):
    raise NotImplementedError("write your pallas kernel here")


# speedup vs baseline: 18.5541x; 18.5541x over previous
import functools

import jax
import jax.numpy as jnp
from jax import lax
from jax.experimental import pallas as pl
from jax.experimental.pallas import tpu as pltpu
from jax.experimental.pallas import tpu_sc as plsc

N = 10000
D = 128
E = 320000
R = 8
RN = R * N

NC = 2
NS = 16
NW = NC * NS
EPT = E // NW

CCH = 400
N_CCH = EPT // CCH

NH = N // NC
EPS = E // NS
SSCH = 400
N_SSCH = EPS // SSCH
SCH = 80
LCAP = EPS + SCH

_mesh = plsc.VectorSubcoreMesh(core_axis_name="c", subcore_axis_name="s")


CB = 80
SLC = RN // NS
SLCP = 5008


@functools.partial(
    pl.kernel,
    out_type=jax.ShapeDtypeStruct((NC * RN,), jnp.float32),
    mesh=_mesh,
    scratch_types=[
        pltpu.VMEM_SHARED((RN,), jnp.float32),
        pltpu.VMEM((CCH,), jnp.int32),
        pltpu.VMEM((CCH,), jnp.int32),
        pltpu.VMEM((CCH // CB, CB), jnp.int32),
        pltpu.VMEM((CB,), jnp.float32),
        pltpu.VMEM((SLCP,), jnp.float32),
    ],
    compiler_params=pltpu.CompilerParams(needs_layout_passes=False),
)
def _count_call(et_hbm, dst_hbm, cnt_hbm, cnt_sh, et_v, d_v, idx_v, one_v,
                stg_v):
    c = lax.axis_index("c")
    s = lax.axis_index("s")
    w = c * NS + s
    one16 = jnp.ones((16,), jnp.float32)
    zero16 = jnp.zeros((16,), jnp.float32)
    for k in range(CB // 16):
        one_v[pl.ds(k * 16, 16)] = one16

    def zb(i, carry):
        stg_v[pl.ds(i * 16, 16)] = zero16
        return carry

    lax.fori_loop(0, SLCP // 16, zb, None)
    pltpu.sync_copy(stg_v.at[pl.ds(0, SLC)], cnt_sh.at[pl.ds(s * SLC, SLC)])
    plsc.subcore_barrier()

    base = w * EPT

    def chunk(i, carry):
        off = base + i * CCH
        pltpu.sync_copy(et_hbm.at[pl.ds(off, CCH)], et_v)
        pltpu.sync_copy(dst_hbm.at[pl.ds(off, CCH)], d_v)

        def grp(k, carry2):
            sl = pl.ds(k * 16, 16)
            row = k // (CB // 16)
            col = (k % (CB // 16)) * 16
            idx_v[row, pl.ds(col, 16)] = et_v[sl] * N + d_v[sl]
            return carry2

        for k in range(CCH // 16):
            grp(k, None)
        for b in range(CCH // CB):
            pltpu.sync_copy(one_v, cnt_sh.at[idx_v.at[b]], add=True)
        return carry

    lax.fori_loop(0, N_CCH, chunk, None)
    plsc.subcore_barrier()
    pltpu.sync_copy(cnt_sh.at[pl.ds(s * SLC, SLC)], stg_v.at[pl.ds(0, SLC)])
    pltpu.sync_copy(stg_v.at[pl.ds(0, SLC)],
                    cnt_hbm.at[pl.ds(c * RN + s * SLC, SLC)])


def _inv_body(cnt_ref, inv_ref):
    total = jnp.sum(cnt_ref[...], axis=0)
    inv_ref[...] = 1.0 / jnp.clip(total, 1.0, None)


_inv_call = pl.pallas_call(
    _inv_body,
    out_shape=jax.ShapeDtypeStruct((625, 128), jnp.float32),
)


_BN = 512
_NB = (N + _BN - 1) // _BN


def _h_body(x_ref, w_ref, h_ref):
    xb = x_ref[...]
    for r in range(R):
        h_ref[r] = jnp.dot(xb, w_ref[r], preferred_element_type=jnp.float32)


_h_call = pl.pallas_call(
    _h_body,
    grid=(_NB,),
    in_specs=[
        pl.BlockSpec((_BN, D), lambda j: (j, 0)),
        pl.BlockSpec((R, D, D), lambda j: (0, 0, 0)),
    ],
    out_specs=pl.BlockSpec((R, _BN, D), lambda j: (0, j, 0)),
    out_shape=jax.ShapeDtypeStruct((R, N, D), jnp.float32),
)


@functools.partial(
    pl.kernel,
    out_type=jax.ShapeDtypeStruct((NC, NH, D), jnp.float32),
    mesh=_mesh,
    scratch_types=[
        pltpu.VMEM_SHARED((NH, D), jnp.float32),
        pltpu.VMEM_SHARED((RN,), jnp.float32),
        pltpu.VMEM((LCAP,), jnp.int32),
        pltpu.VMEM((SSCH,), jnp.int32),
        pltpu.VMEM((SSCH,), jnp.int32),
        pltpu.VMEM((SSCH,), jnp.int32),
        pltpu.VMEM((SCH,), jnp.int32),
        pltpu.VMEM((SCH,), jnp.int32),
        pltpu.VMEM((SCH,), jnp.int32),
        pltpu.VMEM((SCH,), jnp.float32),
        pltpu.VMEM((SCH, D), jnp.float32),
        pltpu.VMEM((RN // NS,), jnp.float32),
        pltpu.SemaphoreType.DMA,
    ],
    compiler_params=pltpu.CompilerParams(needs_layout_passes=False),
)
def _scatter_call(src_hbm, et_hbm, dst_hbm, h_hbm, inv_hbm, part_hbm,
                  acc_sh, inv_sh, list_v, s_v, t_v, d_v,
                  g_v, di_v, ci_v, w_v, rows_v, stg_v, sem):
    c = lax.axis_index("c")
    s = lax.axis_index("s")
    lane = lax.iota(jnp.int32, 16)

    pltpu.sync_copy(inv_hbm.at[pl.ds(s * (RN // NS), RN // NS)], stg_v)
    pltpu.sync_copy(stg_v, inv_sh.at[pl.ds(s * (RN // NS), RN // NS)])

    zero16 = jnp.zeros((16,), jnp.float32)

    def zb(i, carry):
        rows_v[i // 8, pl.ds((i % 8) * 16, 16)] = zero16
        return carry

    lax.fori_loop(0, SCH * 8, zb, None)

    roff = s * 312

    def _for_my_rows(fn):
        @pl.when(s < NS - 1)
        def _():
            for q in range(3):
                fn(roff + q * 80, 80)
            fn(roff + 240, 72)

        @pl.when(s == NS - 1)
        def _():
            for q in range(4):
                fn(roff + q * 80, 80)

    def _zero_rows(row, nrows):
        pltpu.sync_copy(rows_v.at[pl.ds(0, nrows)],
                        acc_sh.at[pl.ds(row, nrows), :])

    _for_my_rows(_zero_rows)

    dlo = c * NH
    base = s * EPS

    def scan_chunk(i, m):
        off = base + i * SSCH
        pltpu.sync_copy(src_hbm.at[pl.ds(off, SSCH)], s_v)
        pltpu.sync_copy(et_hbm.at[pl.ds(off, SSCH)], t_v)
        pltpu.sync_copy(dst_hbm.at[pl.ds(off, SSCH)], d_v)

        def grp(k, m2):
            sl = pl.ds(k * 16, 16)
            d16 = d_v[sl] - dlo
            keep = jnp.logical_and(d16 >= 0, d16 < NH)
            packed = s_v[sl] | (d16 << 14) | (t_v[sl] << 27)
            plsc.store_compressed(list_v.at[pl.ds(m2, 16)], packed, mask=keep)
            nkeep = plsc.all_reduce_population_count(keep)
            return m2 + jnp.max(nkeep)

        return lax.fori_loop(0, SSCH // 16, grp, m)

    m = lax.fori_loop(0, N_SSCH, scan_chunk, jnp.int32(0))

    zpad = jnp.zeros((16,), jnp.int32)
    for k in range(SCH // 16):
        list_v[pl.ds(m + k * 16, 16)] = zpad

    plsc.subcore_barrier()

    h2 = h_hbm
    nb = (m + (SCH - 1)) // SCH

    def flush(b, carry):
        bbase = b * SCH

        def grp(k, carry2):
            sl = pl.ds(k * 16, 16)
            p16 = list_v[pl.ds(bbase + k * 16, 16)]
            s16 = p16 & 0x3FFF
            d16 = (p16 >> 14) & 0x1FFF
            t16 = p16 >> 27
            g_v[sl] = t16 * N + s16
            di_v[sl] = d16
            ci_v[sl] = t16 * N + d16 + dlo
            return carry2

        lax.fori_loop(0, SCH // 16, grp, None)

        cp = pltpu.async_copy(h2.at[g_v], rows_v, sem)
        pltpu.sync_copy(inv_sh.at[ci_v], w_v)
        cp.wait()

        def rowmul(k, carry2):
            valid = (bbase + k * 16 + lane) < m
            w16 = jnp.where(valid, w_v[pl.ds(k * 16, 16)], 0.0)
            for j in range(16):
                wj = w16[j]
                row = k * 16 + j
                for f in range(8):
                    sl = pl.ds(f * 16, 16)
                    rows_v[row, sl] = rows_v[row, sl] * wj
            return carry2

        lax.fori_loop(0, SCH // 16, rowmul, None)

        pltpu.sync_copy(rows_v, acc_sh.at[di_v], add=True)
        return carry

    lax.fori_loop(0, nb, flush, None)
    plsc.subcore_barrier()

    def _out_rows(row, nrows):
        pltpu.sync_copy(acc_sh.at[pl.ds(row, nrows), :],
                        rows_v.at[pl.ds(0, nrows)])
        pltpu.sync_copy(rows_v.at[pl.ds(0, nrows)],
                        part_hbm.at[c, pl.ds(row, nrows), :])

    _for_my_rows(_out_rows)


def _combine_body(x_ref, root_ref, bias_ref, msg_ref, out_ref):
    out_ref[...] = (
        jnp.dot(x_ref[...], root_ref[...], preferred_element_type=jnp.float32)
        + bias_ref[...]
        + msg_ref[...]
    )


_combine_call = pl.pallas_call(
    _combine_body,
    grid=(_NB,),
    in_specs=[
        pl.BlockSpec((_BN, D), lambda j: (j, 0)),
        pl.BlockSpec((D, D), lambda j: (0, 0)),
        pl.BlockSpec((1, D), lambda j: (0, 0)),
        pl.BlockSpec((_BN, D), lambda j: (j, 0)),
    ],
    out_specs=pl.BlockSpec((_BN, D), lambda j: (j, 0)),
    out_shape=jax.ShapeDtypeStruct((N, D), jnp.float32),
)


def kernel(x, edge_index, edge_type, W, root, bias):
    src = edge_index[0].astype(jnp.int32)
    dst = edge_index[1].astype(jnp.int32)
    et = edge_type.astype(jnp.int32)

    cnt = _count_call(et, dst)
    inv = _inv_call(cnt.reshape(NC, 625, 128)).reshape(RN)
    h = _h_call(x, W).reshape(RN, D)
    part = _scatter_call(src, et, dst, h, inv)
    msg = part.reshape(N, D)
    out = _combine_call(x, root, bias.reshape(1, D), msg)
    return (out, edge_index, edge_type)


# speedup vs baseline: 23.5587x; 1.2697x over previous
import functools

import jax
import jax.numpy as jnp
from jax import lax
from jax.experimental import pallas as pl
from jax.experimental.pallas import tpu as pltpu
from jax.experimental.pallas import tpu_sc as plsc

N = 10000
D = 128
E = 320000
R = 8
RN = R * N

NC = 2
NS = 16
NW = NC * NS
EPT = E // NW

CCH = 400
N_CCH = EPT // CCH

NH = N // NC
EPS = E // NS
SSCH = 400
N_SSCH = EPS // SSCH
SCH = 80
LCAP = EPS + SCH

_mesh = plsc.VectorSubcoreMesh(core_axis_name="c", subcore_axis_name="s")


CB = 80
SLC = RN // NS
SLCP = 5008


@functools.partial(
    pl.kernel,
    out_type=jax.ShapeDtypeStruct((NC * RN,), jnp.float32),
    mesh=_mesh,
    scratch_types=[
        pltpu.VMEM_SHARED((RN,), jnp.float32),
        pltpu.VMEM((CCH,), jnp.int32),
        pltpu.VMEM((CCH,), jnp.int32),
        pltpu.VMEM((CCH // CB, CB), jnp.int32),
        pltpu.VMEM((CB,), jnp.float32),
        pltpu.VMEM((SLCP,), jnp.float32),
    ],
    compiler_params=pltpu.CompilerParams(needs_layout_passes=False),
)
def _count_call(et_hbm, dst_hbm, cnt_hbm, cnt_sh, et_v, d_v, idx_v, one_v,
                stg_v):
    c = lax.axis_index("c")
    s = lax.axis_index("s")
    w = c * NS + s
    one16 = jnp.ones((16,), jnp.float32)
    zero16 = jnp.zeros((16,), jnp.float32)
    for k in range(CB // 16):
        one_v[pl.ds(k * 16, 16)] = one16

    def zb(i, carry):
        stg_v[pl.ds(i * 16, 16)] = zero16
        return carry

    lax.fori_loop(0, SLCP // 16, zb, None)
    pltpu.sync_copy(stg_v.at[pl.ds(0, SLC)], cnt_sh.at[pl.ds(s * SLC, SLC)])
    plsc.subcore_barrier()

    base = w * EPT

    def chunk(i, carry):
        off = base + i * CCH
        pltpu.sync_copy(et_hbm.at[pl.ds(off, CCH)], et_v)
        pltpu.sync_copy(dst_hbm.at[pl.ds(off, CCH)], d_v)

        def grp(k, carry2):
            sl = pl.ds(k * 16, 16)
            row = k // (CB // 16)
            col = (k % (CB // 16)) * 16
            idx_v[row, pl.ds(col, 16)] = et_v[sl] * N + d_v[sl]
            return carry2

        for k in range(CCH // 16):
            grp(k, None)
        for b in range(CCH // CB):
            pltpu.sync_copy(one_v, cnt_sh.at[idx_v.at[b]], add=True)
        return carry

    lax.fori_loop(0, N_CCH, chunk, None)
    plsc.subcore_barrier()
    pltpu.sync_copy(cnt_sh.at[pl.ds(s * SLC, SLC)], stg_v.at[pl.ds(0, SLC)])
    pltpu.sync_copy(stg_v.at[pl.ds(0, SLC)],
                    cnt_hbm.at[pl.ds(c * RN + s * SLC, SLC)])


_BN = 512
_NB = (N + _BN - 1) // _BN


def _h_body(x_ref, w_ref, h_ref):
    xb = x_ref[...]
    for r in range(R):
        h_ref[r] = jnp.dot(xb, w_ref[r], preferred_element_type=jnp.float32)


_h_call = pl.pallas_call(
    _h_body,
    grid=(_NB,),
    in_specs=[
        pl.BlockSpec((_BN, D), lambda j: (j, 0)),
        pl.BlockSpec((R, D, D), lambda j: (0, 0, 0)),
    ],
    out_specs=pl.BlockSpec((R, _BN, D), lambda j: (0, j, 0)),
    out_shape=jax.ShapeDtypeStruct((R, N, D), jnp.float32),
)


@functools.partial(
    pl.kernel,
    out_type=jax.ShapeDtypeStruct((NC, NH, D), jnp.float32),
    mesh=_mesh,
    scratch_types=[
        pltpu.VMEM_SHARED((NH, D), jnp.float32),
        pltpu.VMEM_SHARED((RN,), jnp.float32),
        pltpu.VMEM((LCAP,), jnp.int32),
        pltpu.VMEM((SSCH,), jnp.int32),
        pltpu.VMEM((SSCH,), jnp.int32),
        pltpu.VMEM((SSCH,), jnp.int32),
        pltpu.VMEM((SCH,), jnp.int32),
        pltpu.VMEM((SCH,), jnp.int32),
        pltpu.VMEM((SCH,), jnp.int32),
        pltpu.VMEM((SCH,), jnp.float32),
        pltpu.VMEM((SCH, D), jnp.float32),
        pltpu.VMEM((SCH,), jnp.int32),
        pltpu.VMEM((SCH,), jnp.int32),
        pltpu.VMEM((SCH,), jnp.int32),
        pltpu.VMEM((SCH,), jnp.float32),
        pltpu.VMEM((SCH, D), jnp.float32),
        pltpu.VMEM((SLCP,), jnp.float32),
        pltpu.VMEM((SLCP,), jnp.float32),
        pltpu.SemaphoreType.DMA,
        pltpu.SemaphoreType.DMA,
    ],
    compiler_params=pltpu.CompilerParams(needs_layout_passes=False),
)
def _scatter_call(src_hbm, et_hbm, dst_hbm, h_hbm, cnt_hbm, part_hbm,
                  acc_sh, inv_sh, list_v, s_v, t_v, d_v,
                  g0_v, di0_v, ci0_v, w0_v, rows0_v,
                  g1_v, di1_v, ci1_v, w1_v, rows1_v,
                  stg_v, stg2_v, sem0, sem1):
    c = lax.axis_index("c")
    s = lax.axis_index("s")
    lane = lax.iota(jnp.int32, 16)

    pltpu.sync_copy(cnt_hbm.at[pl.ds(s * SLC, SLC)], stg_v.at[pl.ds(0, SLC)])
    pltpu.sync_copy(cnt_hbm.at[pl.ds(RN + s * SLC, SLC)],
                    stg2_v.at[pl.ds(0, SLC)])

    def invb(i, carry):
        sl = pl.ds(i * 16, 16)
        tot = stg_v[sl] + stg2_v[sl]
        stg_v[sl] = 1.0 / jnp.maximum(tot, 1.0)
        return carry

    lax.fori_loop(0, SLC // 16 + 1, invb, None)
    pltpu.sync_copy(stg_v.at[pl.ds(0, SLC)], inv_sh.at[pl.ds(s * SLC, SLC)])

    zero16 = jnp.zeros((16,), jnp.float32)

    def zb(i, carry):
        rows0_v[i // 8, pl.ds((i % 8) * 16, 16)] = zero16
        return carry

    lax.fori_loop(0, SCH * 8, zb, None)

    roff = s * 312

    def _for_my_rows(fn):
        @pl.when(s < NS - 1)
        def _():
            for q in range(3):
                fn(roff + q * 80, 80)
            fn(roff + 240, 72)

        @pl.when(s == NS - 1)
        def _():
            for q in range(4):
                fn(roff + q * 80, 80)

    def _zero_rows(row, nrows):
        pltpu.sync_copy(rows0_v.at[pl.ds(0, nrows)],
                        acc_sh.at[pl.ds(row, nrows), :])

    _for_my_rows(_zero_rows)

    dlo = c * NH
    base = s * EPS

    def scan_chunk(i, m):
        off = base + i * SSCH
        pltpu.sync_copy(src_hbm.at[pl.ds(off, SSCH)], s_v)
        pltpu.sync_copy(et_hbm.at[pl.ds(off, SSCH)], t_v)
        pltpu.sync_copy(dst_hbm.at[pl.ds(off, SSCH)], d_v)

        def grp(k, m2):
            sl = pl.ds(k * 16, 16)
            d16 = d_v[sl] - dlo
            keep = jnp.logical_and(d16 >= 0, d16 < NH)
            packed = s_v[sl] | (d16 << 14) | (t_v[sl] << 27)
            plsc.store_compressed(list_v.at[pl.ds(m2, 16)], packed, mask=keep)
            nkeep = plsc.all_reduce_population_count(keep)
            return m2 + jnp.max(nkeep)

        return lax.fori_loop(0, SSCH // 16, grp, m)

    m = lax.fori_loop(0, N_SSCH, scan_chunk, jnp.int32(0))

    zpad = jnp.zeros((16,), jnp.int32)
    for k in range(SCH // 16):
        list_v[pl.ds(m + k * 16, 16)] = zpad

    plsc.subcore_barrier()

    h2 = h_hbm
    nb = (m + (SCH - 1)) // SCH

    def build_idx(b, g_v, di_v, ci_v):
        bbase = b * SCH

        def grp(k, carry):
            sl = pl.ds(k * 16, 16)
            p16 = list_v[pl.ds(bbase + k * 16, 16)]
            s16 = p16 & 0x3FFF
            d16 = (p16 >> 14) & 0x1FFF
            t16 = p16 >> 27
            g_v[sl] = t16 * N + s16
            di_v[sl] = d16
            ci_v[sl] = t16 * N + d16 + dlo
            return carry

        lax.fori_loop(0, SCH // 16, grp, None)

    def process(b, g_v, di_v, ci_v, w_v, rows_v, sem):
        pltpu.sync_copy(inv_sh.at[ci_v], w_v)
        pltpu.make_async_copy(h2.at[g_v], rows_v, sem).wait()

        def rowmul(k, carry):
            valid = (b * SCH + k * 16 + lane) < m
            w16 = jnp.where(valid, w_v[pl.ds(k * 16, 16)], 0.0)
            for j in range(16):
                wj = w16[j]
                row = k * 16 + j
                for f in range(8):
                    sl = pl.ds(f * 16, 16)
                    rows_v[row, sl] = rows_v[row, sl] * wj
            return carry

        lax.fori_loop(0, SCH // 16, rowmul, None)
        pltpu.sync_copy(rows_v, acc_sh.at[di_v], add=True)

    @pl.when(nb > 0)
    def _():
        build_idx(jnp.int32(0), g0_v, di0_v, ci0_v)
        pltpu.async_copy(h2.at[g0_v], rows0_v, sem0)

    def flush2(i, carry):
        b0 = 2 * i
        b1 = 2 * i + 1

        @pl.when(b1 < nb)
        def _():
            build_idx(b1, g1_v, di1_v, ci1_v)
            pltpu.async_copy(h2.at[g1_v], rows1_v, sem1)

        process(b0, g0_v, di0_v, ci0_v, w0_v, rows0_v, sem0)

        @pl.when(b0 + 2 < nb)
        def _():
            build_idx(b0 + 2, g0_v, di0_v, ci0_v)
            pltpu.async_copy(h2.at[g0_v], rows0_v, sem0)

        @pl.when(b1 < nb)
        def _():
            process(b1, g1_v, di1_v, ci1_v, w1_v, rows1_v, sem1)

        return carry

    lax.fori_loop(0, (nb + 1) // 2, flush2, None)
    plsc.subcore_barrier()

    def _out_rows(row, nrows):
        pltpu.sync_copy(acc_sh.at[pl.ds(row, nrows), :],
                        rows0_v.at[pl.ds(0, nrows)])
        pltpu.sync_copy(rows0_v.at[pl.ds(0, nrows)],
                        part_hbm.at[c, pl.ds(row, nrows), :])

    _for_my_rows(_out_rows)


def _combine_body(x_ref, root_ref, bias_ref, msg_ref, out_ref):
    out_ref[...] = (
        jnp.dot(x_ref[...], root_ref[...], preferred_element_type=jnp.float32)
        + bias_ref[...]
        + msg_ref[...]
    )


_combine_call = pl.pallas_call(
    _combine_body,
    grid=(_NB,),
    in_specs=[
        pl.BlockSpec((_BN, D), lambda j: (j, 0)),
        pl.BlockSpec((D, D), lambda j: (0, 0)),
        pl.BlockSpec((1, D), lambda j: (0, 0)),
        pl.BlockSpec((_BN, D), lambda j: (j, 0)),
    ],
    out_specs=pl.BlockSpec((_BN, D), lambda j: (j, 0)),
    out_shape=jax.ShapeDtypeStruct((N, D), jnp.float32),
)


def kernel(x, edge_index, edge_type, W, root, bias):
    src = edge_index[0].astype(jnp.int32)
    dst = edge_index[1].astype(jnp.int32)
    et = edge_type.astype(jnp.int32)

    cnt = _count_call(et, dst)
    h = _h_call(x, W).reshape(RN, D)
    part = _scatter_call(src, et, dst, h, cnt)
    msg = part.reshape(N, D)
    out = _combine_call(x, root, bias.reshape(1, D), msg)
    return (out, edge_index, edge_type)


# speedup vs baseline: 24.4276x; 1.0369x over previous
import functools

import jax
import jax.numpy as jnp
from jax import lax
from jax.experimental import pallas as pl
from jax.experimental.pallas import tpu as pltpu
from jax.experimental.pallas import tpu_sc as plsc

N = 10000
D = 128
E = 320000
R = 8
RN = R * N

NC = 2
NS = 16

NH = N // NC
EPS = E // NS
SSCH = 400
N_SSCH = EPS // SSCH
SCH = 80
LCAP = EPS + SCH
PADV = R << 27

TBL = 40192
TSLC = TBL // NS

_mesh = plsc.VectorSubcoreMesh(core_axis_name="c", subcore_axis_name="s")


_BN = 512
_NB = (N + _BN - 1) // _BN


def _h_body(x_ref, w_ref, h_ref):
    xb = x_ref[...]
    for r in range(R):
        h_ref[r] = jnp.dot(xb, w_ref[r], preferred_element_type=jnp.float32)


_h_call = pl.pallas_call(
    _h_body,
    grid=(_NB,),
    in_specs=[
        pl.BlockSpec((_BN, D), lambda j: (j, 0)),
        pl.BlockSpec((R, D, D), lambda j: (0, 0, 0)),
    ],
    out_specs=pl.BlockSpec((R, _BN, D), lambda j: (0, j, 0)),
    out_shape=jax.ShapeDtypeStruct((R, N, D), jnp.float32),
)


@functools.partial(
    pl.kernel,
    out_type=jax.ShapeDtypeStruct((NC, NH, D), jnp.float32),
    mesh=_mesh,
    scratch_types=[
        pltpu.VMEM_SHARED((NH, D), jnp.float32),
        pltpu.VMEM_SHARED((TBL,), jnp.float32),
        pltpu.VMEM((LCAP,), jnp.int32),
        pltpu.VMEM((SSCH,), jnp.int32),
        pltpu.VMEM((SSCH,), jnp.int32),
        pltpu.VMEM((SSCH,), jnp.int32),
        pltpu.VMEM((SCH,), jnp.int32),
        pltpu.VMEM((SCH,), jnp.int32),
        pltpu.VMEM((SCH,), jnp.int32),
        pltpu.VMEM((SCH,), jnp.float32),
        pltpu.VMEM((SCH, D), jnp.float32),
        pltpu.VMEM((SCH,), jnp.int32),
        pltpu.VMEM((SCH,), jnp.int32),
        pltpu.VMEM((SCH,), jnp.int32),
        pltpu.VMEM((SCH,), jnp.float32),
        pltpu.VMEM((SCH, D), jnp.float32),
        pltpu.VMEM((SCH,), jnp.float32),
        pltpu.VMEM((TSLC,), jnp.float32),
        pltpu.SemaphoreType.DMA,
        pltpu.SemaphoreType.DMA,
    ],
    compiler_params=pltpu.CompilerParams(needs_layout_passes=False),
)
def _scatter_call(src_hbm, et_hbm, dst_hbm, h_hbm, part_hbm,
                  acc_sh, inv_sh, list_v, s_v, t_v, d_v,
                  g0_v, di0_v, ci0_v, w0_v, rows0_v,
                  g1_v, di1_v, ci1_v, w1_v, rows1_v,
                  one_v, stg_v, sem0, sem1):
    c = lax.axis_index("c")
    s = lax.axis_index("s")
    lane = lax.iota(jnp.int32, 16)
    zero16 = jnp.zeros((16,), jnp.float32)
    one16 = jnp.ones((16,), jnp.float32)

    for k in range(SCH // 16):
        one_v[pl.ds(k * 16, 16)] = one16

    def ztb(i, carry):
        stg_v[pl.ds(i * 16, 16)] = zero16
        return carry

    lax.fori_loop(0, TSLC // 16, ztb, None)
    pltpu.sync_copy(stg_v, inv_sh.at[pl.ds(s * TSLC, TSLC)])

    def zb(i, carry):
        rows0_v[i // 8, pl.ds((i % 8) * 16, 16)] = zero16
        return carry

    lax.fori_loop(0, SCH * 8, zb, None)

    roff = s * 312

    def _for_my_rows(fn):
        @pl.when(s < NS - 1)
        def _():
            for q in range(3):
                fn(roff + q * 80, 80)
            fn(roff + 240, 72)

        @pl.when(s == NS - 1)
        def _():
            for q in range(4):
                fn(roff + q * 80, 80)

    def _zero_rows(row, nrows):
        pltpu.sync_copy(rows0_v.at[pl.ds(0, nrows)],
                        acc_sh.at[pl.ds(row, nrows), :])

    _for_my_rows(_zero_rows)

    dlo = c * NH
    base = s * EPS

    def scan_chunk(i, m):
        off = base + i * SSCH
        pltpu.sync_copy(src_hbm.at[pl.ds(off, SSCH)], s_v)
        pltpu.sync_copy(et_hbm.at[pl.ds(off, SSCH)], t_v)
        pltpu.sync_copy(dst_hbm.at[pl.ds(off, SSCH)], d_v)

        def grp(k, m2):
            sl = pl.ds(k * 16, 16)
            d16 = d_v[sl] - dlo
            keep = jnp.logical_and(d16 >= 0, d16 < NH)
            packed = s_v[sl] | (d16 << 14) | (t_v[sl] << 27)
            plsc.store_compressed(list_v.at[pl.ds(m2, 16)], packed, mask=keep)
            nkeep = plsc.all_reduce_population_count(keep)
            return m2 + jnp.max(nkeep)

        return lax.fori_loop(0, SSCH // 16, grp, m)

    m = lax.fori_loop(0, N_SSCH, scan_chunk, jnp.int32(0))

    pad16 = jnp.full((16,), PADV, jnp.int32)
    for k in range(SCH // 16):
        list_v[pl.ds(m + k * 16, 16)] = pad16

    nb = (m + (SCH - 1)) // SCH
    plsc.subcore_barrier()

    def count_b(b, carry):
        bbase = b * SCH

        def grp(k, carry2):
            sl = pl.ds(k * 16, 16)
            p16 = list_v[pl.ds(bbase + k * 16, 16)]
            d16 = (p16 >> 14) & 0x1FFF
            t16 = (p16 >> 27) & 7
            valid = (bbase + k * 16 + lane) < m
            ci0_v[sl] = jnp.where(valid, t16 * NH + d16, R * NH)
            return carry2

        lax.fori_loop(0, SCH // 16, grp, None)
        pltpu.sync_copy(one_v, inv_sh.at[ci0_v], add=True)
        return carry

    lax.fori_loop(0, nb, count_b, None)
    plsc.subcore_barrier()

    pltpu.sync_copy(inv_sh.at[pl.ds(s * TSLC, TSLC)], stg_v)

    def invb(i, carry):
        sl = pl.ds(i * 16, 16)
        stg_v[sl] = 1.0 / jnp.maximum(stg_v[sl], 1.0)
        return carry

    lax.fori_loop(0, TSLC // 16, invb, None)
    pltpu.sync_copy(stg_v, inv_sh.at[pl.ds(s * TSLC, TSLC)])
    plsc.subcore_barrier()

    h2 = h_hbm

    def build_idx(b, g_v, di_v, ci_v):
        bbase = b * SCH

        def grp(k, carry):
            sl = pl.ds(k * 16, 16)
            p16 = list_v[pl.ds(bbase + k * 16, 16)]
            s16 = p16 & 0x3FFF
            d16 = (p16 >> 14) & 0x1FFF
            t16 = (p16 >> 27) & 7
            g_v[sl] = t16 * N + s16
            di_v[sl] = d16
            ci_v[sl] = t16 * NH + d16
            return carry

        lax.fori_loop(0, SCH // 16, grp, None)

    def process(b, g_v, di_v, ci_v, w_v, rows_v, sem):
        pltpu.sync_copy(inv_sh.at[ci_v], w_v)
        pltpu.make_async_copy(h2.at[g_v], rows_v, sem).wait()

        def rowmul(k, carry):
            valid = (b * SCH + k * 16 + lane) < m
            w16 = jnp.where(valid, w_v[pl.ds(k * 16, 16)], 0.0)
            for j in range(16):
                wj = w16[j]
                row = k * 16 + j
                for f in range(8):
                    sl = pl.ds(f * 16, 16)
                    rows_v[row, sl] = rows_v[row, sl] * wj
            return carry

        lax.fori_loop(0, SCH // 16, rowmul, None)
        pltpu.sync_copy(rows_v, acc_sh.at[di_v], add=True)

    @pl.when(nb > 0)
    def _():
        build_idx(jnp.int32(0), g0_v, di0_v, ci0_v)
        pltpu.async_copy(h2.at[g0_v], rows0_v, sem0)

    def flush2(i, carry):
        b0 = 2 * i
        b1 = 2 * i + 1

        @pl.when(b1 < nb)
        def _():
            build_idx(b1, g1_v, di1_v, ci1_v)
            pltpu.async_copy(h2.at[g1_v], rows1_v, sem1)

        process(b0, g0_v, di0_v, ci0_v, w0_v, rows0_v, sem0)

        @pl.when(b0 + 2 < nb)
        def _():
            build_idx(b0 + 2, g0_v, di0_v, ci0_v)
            pltpu.async_copy(h2.at[g0_v], rows0_v, sem0)

        @pl.when(b1 < nb)
        def _():
            process(b1, g1_v, di1_v, ci1_v, w1_v, rows1_v, sem1)

        return carry

    lax.fori_loop(0, (nb + 1) // 2, flush2, None)
    plsc.subcore_barrier()

    def _out_rows(row, nrows):
        pltpu.sync_copy(acc_sh.at[pl.ds(row, nrows), :],
                        rows0_v.at[pl.ds(0, nrows)])
        pltpu.sync_copy(rows0_v.at[pl.ds(0, nrows)],
                        part_hbm.at[c, pl.ds(row, nrows), :])

    _for_my_rows(_out_rows)


def _combine_body(x_ref, root_ref, bias_ref, msg_ref, out_ref):
    out_ref[...] = (
        jnp.dot(x_ref[...], root_ref[...], preferred_element_type=jnp.float32)
        + bias_ref[...]
        + msg_ref[...]
    )


_combine_call = pl.pallas_call(
    _combine_body,
    grid=(_NB,),
    in_specs=[
        pl.BlockSpec((_BN, D), lambda j: (j, 0)),
        pl.BlockSpec((D, D), lambda j: (0, 0)),
        pl.BlockSpec((1, D), lambda j: (0, 0)),
        pl.BlockSpec((_BN, D), lambda j: (j, 0)),
    ],
    out_specs=pl.BlockSpec((_BN, D), lambda j: (j, 0)),
    out_shape=jax.ShapeDtypeStruct((N, D), jnp.float32),
)


def kernel(x, edge_index, edge_type, W, root, bias):
    src = edge_index[0].astype(jnp.int32)
    dst = edge_index[1].astype(jnp.int32)
    et = edge_type.astype(jnp.int32)

    h = _h_call(x, W).reshape(RN, D)
    part = _scatter_call(src, et, dst, h)
    msg = part.reshape(N, D)
    out = _combine_call(x, root, bias.reshape(1, D), msg)
    return (out, edge_index, edge_type)
